# baseline (speedup 1.0000x reference)
import jax
import jax.numpy as jnp
from jax import lax
from jax.experimental import pallas as pl
from jax.experimental.pallas import tpu as pltpu
from jax.experimental.pallas import tpu_sc as plsc

N = 10000
D = 128
E = 320000
NC, NS = 2, 16
NW = NC * NS
C = 128
NCHUNK = 80
NBUF = 2
NGRP = NCHUNK // NBUF
EPT = NCHUNK * C
E_PAD = EPT * NW
ROWS_PT = 624
TAIL_BASE = ROWS_PT * NS
TAIL_N = N - TAIL_BASE
PAD_ROW = N
SH_ROWS = N + 8
TAIL_Z = SH_ROWS - TAIL_BASE
R = 1000
G = N // R


SH_ROWS_DG = 10016
TAIL_Z_DG = SH_ROWS_DG - TAIL_BASE


def _sc_degree(dst2, zdg, ones_rows):
  scratch = [
      [pltpu.VMEM((C,), jnp.int32) for _ in range(NBUF)],
      pltpu.VMEM((C, D), jnp.float32),
      pltpu.VMEM_SHARED((SH_ROWS_DG, D), jnp.float32),
      pltpu.SemaphoreType.DMA((NBUF,)),
      pltpu.SemaphoreType.DMA((NBUF,)),
  ]
  mesh = plsc.VectorSubcoreMesh(core_axis_name="c", subcore_axis_name="s")

  def body(dst_hbm, zdg_hbm, ones_hbm, deg_out, idst, ones_v, sh_dg,
           isem, ssem):
    c = lax.axis_index("c")
    s = lax.axis_index("s")
    base = (c * NS + s) * NCHUNK
    pltpu.sync_copy(ones_hbm, ones_v)
    pltpu.sync_copy(zdg_hbm, sh_dg.at[pl.ds(s * ROWS_PT, ROWS_PT)])

    @pl.when(s == 0)
    def _():
      pltpu.sync_copy(zdg_hbm.at[pl.ds(0, TAIL_Z_DG)],
                      sh_dg.at[pl.ds(TAIL_BASE, TAIL_Z_DG)])
    plsc.subcore_barrier()

    def fire_idx(j, b):
      pltpu.async_copy(dst_hbm.at[base + j], idst[b], isem.at[b])

    def wait_idx(j, b):
      pltpu.make_async_copy(dst_hbm.at[base + j], idst[b], isem.at[b]).wait()

    for b in range(NBUF):
      fire_idx(b, b)

    def group(g, last):
      descs = []
      for b in range(NBUF):
        wait_idx(g * NBUF + b, b)
        descs.append(pltpu.async_copy(ones_v, sh_dg.at[idst[b]],
                                      ssem.at[b], add=True))
      for b in range(NBUF):
        descs[b].wait()
        if not last:
          fire_idx(g * NBUF + b + NBUF, b)

    @pl.loop(0, NGRP - 1)
    def _(g):
      group(g, last=False)

    group(NGRP - 1, last=True)
    plsc.subcore_barrier()

    pltpu.sync_copy(sh_dg.at[pl.ds(s * ROWS_PT, ROWS_PT)],
                    deg_out.at[c, pl.ds(s * ROWS_PT, ROWS_PT)])

    @pl.when(s == 0)
    def _():
      pltpu.sync_copy(sh_dg.at[pl.ds(TAIL_BASE, TAIL_N)],
                      deg_out.at[c, pl.ds(TAIL_BASE, TAIL_N)])

  fn = pl.kernel(body,
                 out_type=[jax.ShapeDtypeStruct((NC, N, D), jnp.float32)],
                 mesh=mesh, scratch_types=scratch)
  return fn(dst2, zdg, ones_rows)


C2 = 64
NCHUNK2 = EPT // C2
NBUF2 = 4


def _sc_aggregate(h, src1, dst1, z128):
  scratch = [
      pltpu.VMEM((EPT,), jnp.int32),
      [pltpu.VMEM((C2,), jnp.int32) for _ in range(NBUF2)],
      pltpu.VMEM((NBUF2, C2, D), jnp.float32),
      pltpu.VMEM_SHARED((SH_ROWS, D), jnp.float32),
      pltpu.SemaphoreType.DMA((NBUF2,)),
      pltpu.SemaphoreType.DMA((NBUF2,)),
      pltpu.SemaphoreType.DMA((NBUF2,)),
  ]
  mesh = plsc.VectorSubcoreMesh(core_axis_name="c", subcore_axis_name="s")

  def body(h_hbm, src_hbm, dst_hbm, z128_hbm, p_out,
           isrc, idst, rows, sh_p, isem, gsem, ssem):
    c = lax.axis_index("c")
    s = lax.axis_index("s")
    w = c * NS + s
    base = w * EPT

    pltpu.sync_copy(src_hbm.at[pl.ds(base, EPT)], isrc)
    pltpu.sync_copy(z128_hbm, sh_p.at[pl.ds(s * ROWS_PT, ROWS_PT)])

    @pl.when(s == 0)
    def _():
      pltpu.sync_copy(z128_hbm.at[pl.ds(0, TAIL_Z)],
                      sh_p.at[pl.ds(TAIL_BASE, TAIL_Z)])
    plsc.subcore_barrier()

    def fire_idx(j, b):
      pltpu.async_copy(dst_hbm.at[pl.ds(base + j * C2, C2)], idst[b],
                       isem.at[b])

    def wait_idx(j, b):
      pltpu.make_async_copy(dst_hbm.at[pl.ds(base + j * C2, C2)], idst[b],
                            isem.at[b]).wait()

    def fire_gather(j, b):
      pltpu.async_copy(h_hbm.at[isrc.at[pl.ds(j * C2, C2)]], rows.at[b],
                       gsem.at[b])

    def wait_gather(j, b):
      pltpu.make_async_copy(h_hbm.at[isrc.at[pl.ds(j * C2, C2)]], rows.at[b],
                            gsem.at[b]).wait()

    def fire_scatter(b):
      return pltpu.async_copy(rows.at[b], sh_p.at[idst[b]],
                              ssem.at[b], add=True)

    for b in range(NBUF2):
      fire_idx(b, b)
      fire_gather(b, b)

    def group(g, last):
      descs = []
      for b in range(NBUF2):
        j = g * NBUF2 + b
        wait_gather(j, b)
        wait_idx(j, b)
        descs.append(fire_scatter(b))
      for b in range(NBUF2):
        descs[b].wait()
        if not last:
          jn = g * NBUF2 + b + NBUF2
          fire_idx(jn, b)
          fire_gather(jn, b)

    NGRP2 = NCHUNK2 // NBUF2
    @pl.loop(0, NGRP2 - 1)
    def _(g):
      group(g, last=False)

    group(NGRP2 - 1, last=True)
    plsc.subcore_barrier()

    pltpu.sync_copy(sh_p.at[pl.ds(s * ROWS_PT, ROWS_PT)],
                    p_out.at[c, pl.ds(s * ROWS_PT, ROWS_PT)])

    @pl.when(s == 0)
    def _():
      pltpu.sync_copy(sh_p.at[pl.ds(TAIL_BASE, TAIL_N)],
                      p_out.at[c, pl.ds(TAIL_BASE, TAIL_N)])

  fn = pl.kernel(body,
                 out_type=[jax.ShapeDtypeStruct((NC, N, D), jnp.float32)],
                 mesh=mesh, scratch_types=scratch)
  return fn(h, src1, dst1, z128)


def _combine0(h, p0, p1, d0, d1, ws, wn, bias2):
  def body(h_r, p0_r, p1_r, d0_r, d1_r, ws_r, wn_r, b_r, out_r, inv_r):
    deg = (d0_r[...][:, 0:1].astype(jnp.float32)
           + d1_r[...][:, 0:1].astype(jnp.float32))
    inv = 1.0 / jnp.maximum(deg, 1.0)
    inv_r[...] = inv
    mean = (p0_r[...] + p1_r[...]) * inv
    acc = jnp.dot(h_r[...], ws_r[...], preferred_element_type=jnp.float32)
    acc = acc + jnp.dot(mean, wn_r[...], preferred_element_type=jnp.float32)
    acc = acc + b_r[...]
    out_r[...] = jnp.maximum(acc, 0.0)

  return pl.pallas_call(
      body,
      grid=(G,),
      in_specs=[
          pl.BlockSpec((R, D), lambda i: (i, 0)),
          pl.BlockSpec((R, D), lambda i: (i, 0)),
          pl.BlockSpec((R, D), lambda i: (i, 0)),
          pl.BlockSpec((R, D), lambda i: (i, 0)),
          pl.BlockSpec((R, D), lambda i: (i, 0)),
          pl.BlockSpec((D, D), lambda i: (0, 0)),
          pl.BlockSpec((D, D), lambda i: (0, 0)),
          pl.BlockSpec((1, D), lambda i: (0, 0)),
      ],
      out_specs=[
          pl.BlockSpec((R, D), lambda i: (i, 0)),
          pl.BlockSpec((R, 1), lambda i: (i, 0)),
      ],
      out_shape=[
          jax.ShapeDtypeStruct((N, D), jnp.float32),
          jax.ShapeDtypeStruct((N, 1), jnp.float32),
      ],
  )(h, p0, p1, d0, d1, ws, wn, bias2)


def _combine(h, p0, p1, inv, ws, wn, bias2, relu):
  def body(h_r, p0_r, p1_r, inv_r, ws_r, wn_r, b_r, out_r):
    mean = (p0_r[...] + p1_r[...]) * inv_r[...]
    acc = jnp.dot(h_r[...], ws_r[...], preferred_element_type=jnp.float32)
    acc = acc + jnp.dot(mean, wn_r[...], preferred_element_type=jnp.float32)
    acc = acc + b_r[...]
    out_r[...] = jnp.maximum(acc, 0.0) if relu else acc

  return pl.pallas_call(
      body,
      grid=(G,),
      in_specs=[
          pl.BlockSpec((R, D), lambda i: (i, 0)),
          pl.BlockSpec((R, D), lambda i: (i, 0)),
          pl.BlockSpec((R, D), lambda i: (i, 0)),
          pl.BlockSpec((R, 1), lambda i: (i, 0)),
          pl.BlockSpec((D, D), lambda i: (0, 0)),
          pl.BlockSpec((D, D), lambda i: (0, 0)),
          pl.BlockSpec((1, D), lambda i: (0, 0)),
      ],
      out_specs=pl.BlockSpec((R, D), lambda i: (i, 0)),
      out_shape=jax.ShapeDtypeStruct((N, D), jnp.float32),
  )(h, p0, p1, inv, ws, wn, bias2)


def kernel(x, edge_index, W_self0, W_neigh0, b0, W_self1, W_neigh1, b1,
           W_self2, W_neigh2, b2):
  src = edge_index[0].astype(jnp.int32)
  dst = edge_index[1].astype(jnp.int32)
  src1 = jnp.concatenate([src, jnp.zeros((E_PAD - E,), jnp.int32)])
  dst1 = jnp.concatenate([dst, jnp.full((E_PAD - E,), PAD_ROW, jnp.int32)])
  dst2 = dst1.reshape(E_PAD // C, C)
  z128 = jnp.zeros((ROWS_PT, D), jnp.float32)
  zdg = jnp.zeros((ROWS_PT, D), jnp.float32)
  ones_rows = jnp.ones((C, D), jnp.float32)

  (deg,) = _sc_degree(dst2, zdg, ones_rows)
  (p,) = _sc_aggregate(x, src1, dst1, z128)
  h1, inv = _combine0(x, p[0], p[1], deg[0], deg[1], W_self0, W_neigh0,
                      b0.reshape(1, D))
  (p,) = _sc_aggregate(h1, src1, dst1, z128)
  h2 = _combine(h1, p[0], p[1], inv, W_self1, W_neigh1, b1.reshape(1, D),
                relu=True)
  (p,) = _sc_aggregate(h2, src1, dst1, z128)
  h3 = _combine(h2, p[0], p[1], inv, W_self2, W_neigh2, b2.reshape(1, D),
                relu=False)
  return h3


# speedup vs baseline: 1.0484x; 1.0484x over previous
import jax
import jax.numpy as jnp
from jax import lax
from jax.experimental import pallas as pl
from jax.experimental.pallas import tpu as pltpu
from jax.experimental.pallas import tpu_sc as plsc

N = 10000
D = 128
E = 320000
NC, NS = 2, 16
NW = NC * NS
C = 128
NCHUNK = 80
NBUF = 2
NGRP = NCHUNK // NBUF
EPT = NCHUNK * C
E_PAD = EPT * NW
ROWS_PT = 624
TAIL_BASE = ROWS_PT * NS
TAIL_N = N - TAIL_BASE
PAD_ROW = N
SH_ROWS = N + 8
TAIL_Z = SH_ROWS - TAIL_BASE
R = 1000
G = N // R


SH_ROWS_DG = 10016
TAIL_Z_DG = SH_ROWS_DG - TAIL_BASE


def _sc_degree(dst2, zdg, ones_rows):
  scratch = [
      [pltpu.VMEM((C,), jnp.int32) for _ in range(NBUF)],
      pltpu.VMEM((C, D), jnp.float32),
      pltpu.VMEM_SHARED((SH_ROWS_DG, D), jnp.float32),
      pltpu.SemaphoreType.DMA((NBUF,)),
      pltpu.SemaphoreType.DMA((NBUF,)),
  ]
  mesh = plsc.VectorSubcoreMesh(core_axis_name="c", subcore_axis_name="s")

  def body(dst_hbm, zdg_hbm, ones_hbm, deg_out, idst, ones_v, sh_dg,
           isem, ssem):
    c = lax.axis_index("c")
    s = lax.axis_index("s")
    base = (c * NS + s) * NCHUNK
    pltpu.sync_copy(ones_hbm, ones_v)
    pltpu.sync_copy(zdg_hbm, sh_dg.at[pl.ds(s * ROWS_PT, ROWS_PT)])

    @pl.when(s == 0)
    def _():
      pltpu.sync_copy(zdg_hbm.at[pl.ds(0, TAIL_Z_DG)],
                      sh_dg.at[pl.ds(TAIL_BASE, TAIL_Z_DG)])
    plsc.subcore_barrier()

    def fire_idx(j, b):
      pltpu.async_copy(dst_hbm.at[base + j], idst[b], isem.at[b])

    def wait_idx(j, b):
      pltpu.make_async_copy(dst_hbm.at[base + j], idst[b], isem.at[b]).wait()

    for b in range(NBUF):
      fire_idx(b, b)

    def group(g, last):
      descs = []
      for b in range(NBUF):
        wait_idx(g * NBUF + b, b)
        descs.append(pltpu.async_copy(ones_v, sh_dg.at[idst[b]],
                                      ssem.at[b], add=True))
      for b in range(NBUF):
        descs[b].wait()
        if not last:
          fire_idx(g * NBUF + b + NBUF, b)

    @pl.loop(0, NGRP - 1)
    def _(g):
      group(g, last=False)

    group(NGRP - 1, last=True)
    plsc.subcore_barrier()

    pltpu.sync_copy(sh_dg.at[pl.ds(s * ROWS_PT, ROWS_PT)],
                    deg_out.at[c, pl.ds(s * ROWS_PT, ROWS_PT)])

    @pl.when(s == 0)
    def _():
      pltpu.sync_copy(sh_dg.at[pl.ds(TAIL_BASE, TAIL_N)],
                      deg_out.at[c, pl.ds(TAIL_BASE, TAIL_N)])

  fn = pl.kernel(body,
                 out_type=[jax.ShapeDtypeStruct((NC, N, D), jnp.float32)],
                 mesh=mesh, scratch_types=scratch)
  return fn(dst2, zdg, ones_rows)


def _sc_aggregate(h, src2, dst2, z128):
  scratch = [
      pltpu.VMEM((NCHUNK, C), jnp.int32),
      [pltpu.VMEM((C,), jnp.int32) for _ in range(NBUF)],
      pltpu.VMEM((NBUF, C, D), jnp.float32),
      pltpu.VMEM_SHARED((SH_ROWS, D), jnp.float32),
      pltpu.SemaphoreType.DMA((NBUF,)),
      pltpu.SemaphoreType.DMA((NBUF,)),
      pltpu.SemaphoreType.DMA((NBUF,)),
  ]
  mesh = plsc.VectorSubcoreMesh(core_axis_name="c", subcore_axis_name="s")

  def body(h_hbm, src_hbm, dst_hbm, z128_hbm, p_out,
           isrc, idst, rows, sh_p, isem, gsem, ssem):
    c = lax.axis_index("c")
    s = lax.axis_index("s")
    w = c * NS + s
    base = w * NCHUNK

    pltpu.sync_copy(src_hbm.at[pl.ds(base, NCHUNK)], isrc)
    pltpu.sync_copy(z128_hbm, sh_p.at[pl.ds(s * ROWS_PT, ROWS_PT)])

    @pl.when(s == 0)
    def _():
      pltpu.sync_copy(z128_hbm.at[pl.ds(0, TAIL_Z)],
                      sh_p.at[pl.ds(TAIL_BASE, TAIL_Z)])
    plsc.subcore_barrier()

    def fire_idx(j, b):
      pltpu.async_copy(dst_hbm.at[base + j], idst[b], isem.at[b])

    def wait_idx(j, b):
      pltpu.make_async_copy(dst_hbm.at[base + j], idst[b], isem.at[b]).wait()

    def fire_gather(j, b):
      pltpu.async_copy(h_hbm.at[isrc.at[j]], rows.at[b], gsem.at[b])

    def wait_gather(j, b):
      pltpu.make_async_copy(h_hbm.at[isrc.at[j]], rows.at[b],
                            gsem.at[b]).wait()

    def fire_scatter(b):
      return pltpu.async_copy(rows.at[b], sh_p.at[idst[b]],
                              ssem.at[b], add=True)

    for b in range(NBUF):
      fire_idx(b, b)
      fire_gather(b, b)

    def group(g, last):
      descs = []
      for b in range(NBUF):
        j = g * NBUF + b
        wait_gather(j, b)
        wait_idx(j, b)
        descs.append(fire_scatter(b))
      for b in range(NBUF):
        descs[b].wait()
        if not last:
          jn = g * NBUF + b + NBUF
          fire_idx(jn, b)
          fire_gather(jn, b)

    @pl.loop(0, NGRP - 1)
    def _(g):
      group(g, last=False)

    group(NGRP - 1, last=True)
    plsc.subcore_barrier()

    pltpu.sync_copy(sh_p.at[pl.ds(s * ROWS_PT, ROWS_PT)],
                    p_out.at[c, pl.ds(s * ROWS_PT, ROWS_PT)])

    @pl.when(s == 0)
    def _():
      pltpu.sync_copy(sh_p.at[pl.ds(TAIL_BASE, TAIL_N)],
                      p_out.at[c, pl.ds(TAIL_BASE, TAIL_N)])

  fn = pl.kernel(body,
                 out_type=[jax.ShapeDtypeStruct((NC, N, D), jnp.float32)],
                 mesh=mesh, scratch_types=scratch)
  return fn(h, src2, dst2, z128)


def _combine0(h, p0, p1, d0, d1, ws, wn, bias2):
  def body(h_r, p0_r, p1_r, d0_r, d1_r, ws_r, wn_r, b_r, out_r, inv_r):
    deg = (d0_r[...][:, 0:1].astype(jnp.float32)
           + d1_r[...][:, 0:1].astype(jnp.float32))
    inv = 1.0 / jnp.maximum(deg, 1.0)
    inv_r[...] = inv
    mean = (p0_r[...] + p1_r[...]) * inv
    acc = jnp.dot(h_r[...], ws_r[...], preferred_element_type=jnp.float32)
    acc = acc + jnp.dot(mean, wn_r[...], preferred_element_type=jnp.float32)
    acc = acc + b_r[...]
    out_r[...] = jnp.maximum(acc, 0.0)

  return pl.pallas_call(
      body,
      grid=(G,),
      in_specs=[
          pl.BlockSpec((R, D), lambda i: (i, 0)),
          pl.BlockSpec((R, D), lambda i: (i, 0)),
          pl.BlockSpec((R, D), lambda i: (i, 0)),
          pl.BlockSpec((R, D), lambda i: (i, 0)),
          pl.BlockSpec((R, D), lambda i: (i, 0)),
          pl.BlockSpec((D, D), lambda i: (0, 0)),
          pl.BlockSpec((D, D), lambda i: (0, 0)),
          pl.BlockSpec((1, D), lambda i: (0, 0)),
      ],
      out_specs=[
          pl.BlockSpec((R, D), lambda i: (i, 0)),
          pl.BlockSpec((R, 1), lambda i: (i, 0)),
      ],
      out_shape=[
          jax.ShapeDtypeStruct((N, D), jnp.float32),
          jax.ShapeDtypeStruct((N, 1), jnp.float32),
      ],
  )(h, p0, p1, d0, d1, ws, wn, bias2)


def _combine(h, p0, p1, inv, ws, wn, bias2, relu):
  def body(h_r, p0_r, p1_r, inv_r, ws_r, wn_r, b_r, out_r):
    mean = (p0_r[...] + p1_r[...]) * inv_r[...]
    acc = jnp.dot(h_r[...], ws_r[...], preferred_element_type=jnp.float32)
    acc = acc + jnp.dot(mean, wn_r[...], preferred_element_type=jnp.float32)
    acc = acc + b_r[...]
    out_r[...] = jnp.maximum(acc, 0.0) if relu else acc

  return pl.pallas_call(
      body,
      grid=(G,),
      in_specs=[
          pl.BlockSpec((R, D), lambda i: (i, 0)),
          pl.BlockSpec((R, D), lambda i: (i, 0)),
          pl.BlockSpec((R, D), lambda i: (i, 0)),
          pl.BlockSpec((R, 1), lambda i: (i, 0)),
          pl.BlockSpec((D, D), lambda i: (0, 0)),
          pl.BlockSpec((D, D), lambda i: (0, 0)),
          pl.BlockSpec((1, D), lambda i: (0, 0)),
      ],
      out_specs=pl.BlockSpec((R, D), lambda i: (i, 0)),
      out_shape=jax.ShapeDtypeStruct((N, D), jnp.float32),
  )(h, p0, p1, inv, ws, wn, bias2)


def kernel(x, edge_index, W_self0, W_neigh0, b0, W_self1, W_neigh1, b1,
           W_self2, W_neigh2, b2):
  src = edge_index[0].astype(jnp.int32)
  dst = edge_index[1].astype(jnp.int32)
  src2 = jnp.concatenate(
      [src, jnp.zeros((E_PAD - E,), jnp.int32)]).reshape(E_PAD // C, C)
  dst2 = jnp.concatenate(
      [dst, jnp.full((E_PAD - E,), PAD_ROW, jnp.int32)]).reshape(E_PAD // C, C)
  z128 = jnp.zeros((ROWS_PT, D), jnp.float32)
  zdg = jnp.zeros((ROWS_PT, D), jnp.float32)
  ones_rows = jnp.ones((C, D), jnp.float32)

  (deg,) = _sc_degree(dst2, zdg, ones_rows)
  (p,) = _sc_aggregate(x, src2, dst2, z128)
  h1, inv = _combine0(x, p[0], p[1], deg[0], deg[1], W_self0, W_neigh0,
                      b0.reshape(1, D))
  (p,) = _sc_aggregate(h1, src2, dst2, z128)
  h2 = _combine(h1, p[0], p[1], inv, W_self1, W_neigh1, b1.reshape(1, D),
                relu=True)
  (p,) = _sc_aggregate(h2, src2, dst2, z128)
  h3 = _combine(h2, p[0], p[1], inv, W_self2, W_neigh2, b2.reshape(1, D),
                relu=False)
  return h3
